# speedup vs baseline: 1.2148x; 1.2148x over previous
import jax
import jax.numpy as jnp
import numpy as np
from jax import lax
from jax.experimental import pallas as pl
from jax.experimental.pallas import tpu as pltpu
from jax.experimental.pallas import tpu_sc as plsc

VOCAB = 100000
D_MODEL = 128
SEQ_LEN = 2048
BATCH = 4

_NC = 2
_NS = 16
_NW = _NC * _NS
_B_FLAT = BATCH * SEQ_LEN
_CHUNK = _B_FLAT // _NW
_DV = D_MODEL // 16


def _sinusoidal_pe() -> np.ndarray:
    pos = np.arange(SEQ_LEN)[:, None].astype(np.float64)
    i = np.arange(D_MODEL)[None, :].astype(np.float64)
    angle = pos / np.power(10000.0, (2.0 * (i // 2)) / D_MODEL)
    pe = np.zeros((SEQ_LEN, D_MODEL), dtype=np.float32)
    pe[:, 0::2] = np.sin(angle[:, 0::2])
    pe[:, 1::2] = np.cos(angle[:, 1::2])
    return pe


_PE = _sinusoidal_pe()


def _embed_body(w_hbm, x1_hbm, x2_hbm, pe_hbm, out1_hbm, out2_hbm,
                idx1_v, idx2_v, pe_v, rows1_v, rows2_v,
                sem_pe, sem1, sem2):
    wid = lax.axis_index("s") * _NC + lax.axis_index("c")
    base = wid * _CHUNK
    s0 = lax.rem(base, SEQ_LEN)

    cp_pe = pltpu.async_copy(pe_hbm.at[pl.ds(s0, _CHUNK)], pe_v, sem_pe)
    pltpu.sync_copy(x1_hbm.at[pl.ds(base, _CHUNK)], idx1_v)
    pltpu.sync_copy(x2_hbm.at[pl.ds(base, _CHUNK)], idx2_v)
    cp1 = pltpu.async_copy(w_hbm.at[idx1_v], rows1_v, sem1)
    cp2 = pltpu.async_copy(w_hbm.at[idx2_v], rows2_v, sem2)

    def add_pe(rows_ref):
        def body(r, carry):
            for d in range(_DV):
                sl = pl.ds(d * 16, 16)
                rows_ref[r, sl] = rows_ref[r, sl] + pe_v[r, sl]
            return carry
        lax.fori_loop(0, _CHUNK, body, 0)

    cp_pe.wait()
    cp1.wait()
    add_pe(rows1_v)
    pltpu.sync_copy(rows1_v, out1_hbm.at[pl.ds(base, _CHUNK)])
    cp2.wait()
    add_pe(rows2_v)
    pltpu.sync_copy(rows2_v, out2_hbm.at[pl.ds(base, _CHUNK)])


_sc_embed = pl.kernel(
    _embed_body,
    out_type=(
        jax.ShapeDtypeStruct((_B_FLAT, D_MODEL), jnp.float32),
        jax.ShapeDtypeStruct((_B_FLAT, D_MODEL), jnp.float32),
    ),
    mesh=plsc.VectorSubcoreMesh(core_axis_name="c", subcore_axis_name="s"),
    scratch_types=[
        pltpu.VMEM((_CHUNK,), jnp.int32),
        pltpu.VMEM((_CHUNK,), jnp.int32),
        pltpu.VMEM((_CHUNK, D_MODEL), jnp.float32),
        pltpu.VMEM((_CHUNK, D_MODEL), jnp.float32),
        pltpu.VMEM((_CHUNK, D_MODEL), jnp.float32),
        pltpu.SemaphoreType.DMA,
        pltpu.SemaphoreType.DMA,
        pltpu.SemaphoreType.DMA,
    ],
)


@jax.jit
def kernel(x, x_output, W):
    pe = jnp.asarray(_PE)
    h1, h2 = _sc_embed(W, x.reshape(-1), x_output.reshape(-1), pe)
    return (h1.reshape(BATCH, SEQ_LEN, D_MODEL),
            h2.reshape(BATCH, SEQ_LEN, D_MODEL))


# speedup vs baseline: 1.2527x; 1.0312x over previous
import jax
import jax.numpy as jnp
import numpy as np
from jax import lax
from jax.experimental import pallas as pl
from jax.experimental.pallas import tpu as pltpu
from jax.experimental.pallas import tpu_sc as plsc

VOCAB = 100000
D_MODEL = 128
SEQ_LEN = 2048
BATCH = 4

_NC = 2
_NS = 16
_NW = _NC * _NS
_B_FLAT = BATCH * SEQ_LEN
_CHUNK = _B_FLAT // _NW
_DV = D_MODEL // 16


def _sinusoidal_pe() -> np.ndarray:
    pos = np.arange(SEQ_LEN)[:, None].astype(np.float64)
    i = np.arange(D_MODEL)[None, :].astype(np.float64)
    angle = pos / np.power(10000.0, (2.0 * (i // 2)) / D_MODEL)
    pe = np.zeros((SEQ_LEN, D_MODEL), dtype=np.float32)
    pe[:, 0::2] = np.sin(angle[:, 0::2])
    pe[:, 1::2] = np.cos(angle[:, 1::2])
    return pe


_PE = _sinusoidal_pe()


def _embed_body(w_hbm, x1_hbm, x2_hbm, pe_hbm, out1_hbm, out2_hbm,
                idx1_v, idx2_v, pe_v, rows1_v, rows2_v,
                sem_pe, sem1, sem2, sem_o1, sem_o2):
    wid = lax.axis_index("s") * _NC + lax.axis_index("c")
    base = wid * _CHUNK
    s0 = lax.rem(base, SEQ_LEN)

    cp_pe = pltpu.async_copy(pe_hbm.at[pl.ds(s0, _CHUNK)], pe_v, sem_pe)
    pltpu.sync_copy(x1_hbm.at[pl.ds(base, _CHUNK)], idx1_v)
    pltpu.sync_copy(x2_hbm.at[pl.ds(base, _CHUNK)], idx2_v)
    cp1 = pltpu.async_copy(w_hbm.at[idx1_v], rows1_v, sem1)
    cp2 = pltpu.async_copy(w_hbm.at[idx2_v], rows2_v, sem2)

    def add_pe(rows_ref):
        @plsc.parallel_loop(0, _CHUNK, step=2)
        def _(r):
            for rr in range(2):
                for d in range(_DV):
                    sl = pl.ds(d * 16, 16)
                    plsc.addupdate(rows_ref.at[r + rr, sl], pe_v[r + rr, sl])

    cp_pe.wait()
    cp1.wait()
    add_pe(rows1_v)
    st1 = pltpu.async_copy(rows1_v, out1_hbm.at[pl.ds(base, _CHUNK)], sem_o1)
    cp2.wait()
    add_pe(rows2_v)
    st2 = pltpu.async_copy(rows2_v, out2_hbm.at[pl.ds(base, _CHUNK)], sem_o2)
    st1.wait()
    st2.wait()


_sc_embed = pl.kernel(
    _embed_body,
    out_type=(
        jax.ShapeDtypeStruct((_B_FLAT, D_MODEL), jnp.float32),
        jax.ShapeDtypeStruct((_B_FLAT, D_MODEL), jnp.float32),
    ),
    mesh=plsc.VectorSubcoreMesh(core_axis_name="c", subcore_axis_name="s"),
    scratch_types=[
        pltpu.VMEM((_CHUNK,), jnp.int32),
        pltpu.VMEM((_CHUNK,), jnp.int32),
        pltpu.VMEM((_CHUNK, D_MODEL), jnp.float32),
        pltpu.VMEM((_CHUNK, D_MODEL), jnp.float32),
        pltpu.VMEM((_CHUNK, D_MODEL), jnp.float32),
        pltpu.SemaphoreType.DMA,
        pltpu.SemaphoreType.DMA,
        pltpu.SemaphoreType.DMA,
        pltpu.SemaphoreType.DMA,
        pltpu.SemaphoreType.DMA,
    ],
)


@jax.jit
def kernel(x, x_output, W):
    pe = jnp.asarray(_PE)
    h1, h2 = _sc_embed(W, x.reshape(-1), x_output.reshape(-1), pe)
    return (h1.reshape(BATCH, SEQ_LEN, D_MODEL),
            h2.reshape(BATCH, SEQ_LEN, D_MODEL))


# speedup vs baseline: 1.3097x; 1.0455x over previous
import jax
import jax.numpy as jnp
import numpy as np
from jax import lax
from jax.experimental import pallas as pl
from jax.experimental.pallas import tpu as pltpu
from jax.experimental.pallas import tpu_sc as plsc

VOCAB = 100000
D_MODEL = 128
SEQ_LEN = 2048
BATCH = 4

_NC = 2
_NS = 16
_NW = _NC * _NS
_B_FLAT = BATCH * SEQ_LEN
_CHUNK = _B_FLAT // _NW
_DV = D_MODEL // 16


def _sinusoidal_pe() -> np.ndarray:
    pos = np.arange(SEQ_LEN)[:, None].astype(np.float64)
    i = np.arange(D_MODEL)[None, :].astype(np.float64)
    angle = pos / np.power(10000.0, (2.0 * (i // 2)) / D_MODEL)
    pe = np.zeros((SEQ_LEN, D_MODEL), dtype=np.float32)
    pe[:, 0::2] = np.sin(angle[:, 0::2])
    pe[:, 1::2] = np.cos(angle[:, 1::2])
    return pe


_PE = _sinusoidal_pe()


def _embed_body(w_hbm, x1_hbm, x2_hbm, pe_hbm, out1_hbm, out2_hbm,
                idx1_v, idx2_v, pe_v, rows1_v, rows2_v,
                sem_pe, sem1, sem2, sem_o1, sem_o2):
    wid = lax.axis_index("s") * _NC + lax.axis_index("c")
    base = wid * _CHUNK
    b = base // SEQ_LEN
    s0 = lax.rem(base, SEQ_LEN)

    cp_pe = pltpu.async_copy(pe_hbm.at[pl.ds(s0, _CHUNK)], pe_v, sem_pe)
    pltpu.sync_copy(x1_hbm.at[b, pl.ds(s0, _CHUNK)], idx1_v)
    pltpu.sync_copy(x2_hbm.at[b, pl.ds(s0, _CHUNK)], idx2_v)
    cp1 = pltpu.async_copy(w_hbm.at[idx1_v], rows1_v, sem1)
    cp2 = pltpu.async_copy(w_hbm.at[idx2_v], rows2_v, sem2)

    def add_pe(rows_ref):
        @plsc.parallel_loop(0, _CHUNK, step=2)
        def _(r):
            for rr in range(2):
                for d in range(_DV):
                    sl = pl.ds(d * 16, 16)
                    plsc.addupdate(rows_ref.at[r + rr, sl], pe_v[r + rr, sl])

    cp_pe.wait()
    cp1.wait()
    add_pe(rows1_v)
    st1 = pltpu.async_copy(rows1_v, out1_hbm.at[b, pl.ds(s0, _CHUNK)], sem_o1)
    cp2.wait()
    add_pe(rows2_v)
    st2 = pltpu.async_copy(rows2_v, out2_hbm.at[b, pl.ds(s0, _CHUNK)], sem_o2)
    st1.wait()
    st2.wait()


_sc_embed = pl.kernel(
    _embed_body,
    out_type=(
        jax.ShapeDtypeStruct((BATCH, SEQ_LEN, D_MODEL), jnp.float32),
        jax.ShapeDtypeStruct((BATCH, SEQ_LEN, D_MODEL), jnp.float32),
    ),
    mesh=plsc.VectorSubcoreMesh(core_axis_name="c", subcore_axis_name="s"),
    scratch_types=[
        pltpu.VMEM((_CHUNK,), jnp.int32),
        pltpu.VMEM((_CHUNK,), jnp.int32),
        pltpu.VMEM((_CHUNK, D_MODEL), jnp.float32),
        pltpu.VMEM((_CHUNK, D_MODEL), jnp.float32),
        pltpu.VMEM((_CHUNK, D_MODEL), jnp.float32),
        pltpu.SemaphoreType.DMA,
        pltpu.SemaphoreType.DMA,
        pltpu.SemaphoreType.DMA,
        pltpu.SemaphoreType.DMA,
        pltpu.SemaphoreType.DMA,
    ],
)


@jax.jit
def kernel(x, x_output, W):
    pe = jnp.asarray(_PE)
    return _sc_embed(W, x, x_output, pe)


# speedup vs baseline: 1.3104x; 1.0006x over previous
import jax
import jax.numpy as jnp
import numpy as np
from jax import lax
from jax.experimental import pallas as pl
from jax.experimental.pallas import tpu as pltpu
from jax.experimental.pallas import tpu_sc as plsc

VOCAB = 100000
D_MODEL = 128
SEQ_LEN = 2048
BATCH = 4

_NC = 2
_NS = 16
_NW = _NC * _NS
_B_FLAT = BATCH * SEQ_LEN
_CHUNK = _B_FLAT // _NW
_DV = D_MODEL // 16


def _sinusoidal_pe() -> np.ndarray:
    pos = np.arange(SEQ_LEN)[:, None].astype(np.float64)
    i = np.arange(D_MODEL)[None, :].astype(np.float64)
    angle = pos / np.power(10000.0, (2.0 * (i // 2)) / D_MODEL)
    pe = np.zeros((SEQ_LEN, D_MODEL), dtype=np.float32)
    pe[:, 0::2] = np.sin(angle[:, 0::2])
    pe[:, 1::2] = np.cos(angle[:, 1::2])
    return pe


_PE = _sinusoidal_pe()


def _embed_body(w_hbm, x1_hbm, x2_hbm, pe_hbm, out1_hbm, out2_hbm,
                idx1_v, idx2_v, pe_v, rows1_v, rows2_v,
                sem_pe, sem1, sem2, sem_o1, sem_o2):
    wid = lax.axis_index("s") * _NC + lax.axis_index("c")
    base = wid * _CHUNK
    b = base // SEQ_LEN
    s0 = lax.rem(base, SEQ_LEN)

    cp_pe = pltpu.async_copy(
        pe_hbm.at[pl.ds(s0 * D_MODEL, _CHUNK * D_MODEL)], pe_v, sem_pe)
    pltpu.sync_copy(x1_hbm.at[b, pl.ds(s0, _CHUNK)], idx1_v)
    pltpu.sync_copy(x2_hbm.at[b, pl.ds(s0, _CHUNK)], idx2_v)
    cp1 = pltpu.async_copy(w_hbm.at[idx1_v], rows1_v, sem1)
    cp2 = pltpu.async_copy(w_hbm.at[idx2_v], rows2_v, sem2)

    def add_pe(rows_ref):
        @plsc.parallel_loop(0, _CHUNK, step=2)
        def _(r):
            for rr in range(2):
                for d in range(_DV):
                    sl = pl.ds(d * 16, 16)
                    pv = pe_v[pl.ds((r + rr) * D_MODEL + d * 16, 16)]
                    plsc.addupdate(rows_ref.at[r + rr, sl], pv)

    cp_pe.wait()
    cp1.wait()
    add_pe(rows1_v)
    st1 = pltpu.async_copy(rows1_v, out1_hbm.at[b, pl.ds(s0, _CHUNK)], sem_o1)
    cp2.wait()
    add_pe(rows2_v)
    st2 = pltpu.async_copy(rows2_v, out2_hbm.at[b, pl.ds(s0, _CHUNK)], sem_o2)
    st1.wait()
    st2.wait()


_sc_embed = pl.kernel(
    _embed_body,
    out_type=(
        jax.ShapeDtypeStruct((BATCH, SEQ_LEN, D_MODEL), jnp.float32),
        jax.ShapeDtypeStruct((BATCH, SEQ_LEN, D_MODEL), jnp.float32),
    ),
    mesh=plsc.VectorSubcoreMesh(core_axis_name="c", subcore_axis_name="s"),
    scratch_types=[
        pltpu.VMEM((_CHUNK,), jnp.int32),
        pltpu.VMEM((_CHUNK,), jnp.int32),
        pltpu.VMEM((_CHUNK * D_MODEL,), jnp.float32),
        pltpu.VMEM((_CHUNK, D_MODEL), jnp.float32),
        pltpu.VMEM((_CHUNK, D_MODEL), jnp.float32),
        pltpu.SemaphoreType.DMA,
        pltpu.SemaphoreType.DMA,
        pltpu.SemaphoreType.DMA,
        pltpu.SemaphoreType.DMA,
        pltpu.SemaphoreType.DMA,
    ],
)


@jax.jit
def kernel(x, x_output, W):
    pe = jnp.asarray(_PE.reshape(-1))
    return _sc_embed(W, x, x_output, pe)


# speedup vs baseline: 1.3556x; 1.0345x over previous
import jax
import jax.numpy as jnp
import numpy as np
from jax import lax
from jax.experimental import pallas as pl
from jax.experimental.pallas import tpu as pltpu
from jax.experimental.pallas import tpu_sc as plsc

VOCAB = 100000
D_MODEL = 128
SEQ_LEN = 2048
BATCH = 4

_NC = 2
_NS = 16
_NW = _NC * _NS
_B_FLAT = BATCH * SEQ_LEN
_CHUNK = _B_FLAT // _NW
_DV = D_MODEL // 16


def _sinusoidal_pe() -> np.ndarray:
    pos = np.arange(SEQ_LEN)[:, None].astype(np.float64)
    i = np.arange(D_MODEL)[None, :].astype(np.float64)
    angle = pos / np.power(10000.0, (2.0 * (i // 2)) / D_MODEL)
    pe = np.zeros((SEQ_LEN, D_MODEL), dtype=np.float32)
    pe[:, 0::2] = np.sin(angle[:, 0::2])
    pe[:, 1::2] = np.cos(angle[:, 1::2])
    return pe


_PE = _sinusoidal_pe()


_NSUB = 4
_SUB = _CHUNK // _NSUB


def _embed_body(w_hbm, x1_hbm, x2_hbm, pe_hbm, out1_hbm, out2_hbm,
                idx1_v, idx2_v, pe_v, rows1_v, rows2_v,
                sem_pe, *sems):
    wid = lax.axis_index("s") * _NC + lax.axis_index("c")
    base = wid * _CHUNK
    b = base // SEQ_LEN
    s0 = lax.rem(base, SEQ_LEN)

    cp_pe = pltpu.async_copy(
        pe_hbm.at[pl.ds(s0 * D_MODEL, _CHUNK * D_MODEL)], pe_v, sem_pe)
    pltpu.sync_copy(x1_hbm.at[b, pl.ds(s0, _CHUNK)], idx1_v)
    pltpu.sync_copy(x2_hbm.at[b, pl.ds(s0, _CHUNK)], idx2_v)

    gathers = []
    for j in range(_NSUB):
        sl = pl.ds(_SUB * j, _SUB)
        for idx_v, rows_v in ((idx1_v, rows1_v), (idx2_v, rows2_v)):
            sem = sems[len(gathers)]
            gathers.append(
                (pltpu.async_copy(w_hbm.at[idx_v.at[sl]], rows_v.at[sl], sem),
                 sem))

    def add_pe(rows_ref, j):
        @plsc.parallel_loop(_SUB * j, _SUB * (j + 1), step=2)
        def _(r):
            for rr in range(2):
                for d in range(_DV):
                    sl = pl.ds(d * 16, 16)
                    pv = pe_v[pl.ds((r + rr) * D_MODEL + d * 16, 16)]
                    plsc.addupdate(rows_ref.at[r + rr, sl], pv)

    cp_pe.wait()
    stores = []
    for j in range(_NSUB):
        sl = pl.ds(_SUB * j, _SUB)
        for k, (rows_v, out_hbm) in enumerate(((rows1_v, out1_hbm),
                                               (rows2_v, out2_hbm))):
            cp, sem = gathers[2 * j + k]
            cp.wait()
            add_pe(rows_v, j)
            o0 = s0 + _SUB * j
            stores.append(pltpu.async_copy(
                rows_v.at[sl], out_hbm.at[b, pl.ds(o0, _SUB)], sem))
    for st in stores:
        st.wait()


_sc_embed = pl.kernel(
    _embed_body,
    out_type=(
        jax.ShapeDtypeStruct((BATCH, SEQ_LEN, D_MODEL), jnp.float32),
        jax.ShapeDtypeStruct((BATCH, SEQ_LEN, D_MODEL), jnp.float32),
    ),
    mesh=plsc.VectorSubcoreMesh(core_axis_name="c", subcore_axis_name="s"),
    scratch_types=[
        pltpu.VMEM((_CHUNK,), jnp.int32),
        pltpu.VMEM((_CHUNK,), jnp.int32),
        pltpu.VMEM((_CHUNK * D_MODEL,), jnp.float32),
        pltpu.VMEM((_CHUNK, D_MODEL), jnp.float32),
        pltpu.VMEM((_CHUNK, D_MODEL), jnp.float32),
        pltpu.SemaphoreType.DMA,
    ] + [pltpu.SemaphoreType.DMA] * (2 * _NSUB),
)


@jax.jit
def kernel(x, x_output, W):
    pe = jnp.asarray(_PE.reshape(-1))
    return _sc_embed(W, x, x_output, pe)
